# baseline (speedup 1.0000x reference)
import functools

import jax
import jax.numpy as jnp
import numpy as np
from jax.experimental import pallas as pl
from jax.experimental.pallas import tpu as pltpu

_PI = float(np.pi)
_BF = jnp.bfloat16


def _mlpe_block(x_ref, lat_ref, lon_ref, sst_ref, date_ref,
                W1_ref, b1_ref, rb1_ref, W2_ref, b2_ref, rb2_ref,
                W3_ref, b3_ref, rb3_ref, W4_ref, b4_ref, out_ref,
                T1_s, T3_s, W2_s, W3b_s, W4_s, c1_s, c2_s, c3_s, b4_s):
    f32 = jnp.float32
    dot = functools.partial(jnp.dot, preferred_element_type=f32)
    dot_hi = functools.partial(jnp.dot, precision=jax.lax.Precision.HIGHEST,
                               preferred_element_type=f32)

    k = float(np.sqrt(_PI / np.log(2.0)))

    @pl.when(pl.program_id(0) == 0)
    def _fold():
        def tmat(W_ref):
            W = W_ref[0:68, :]
            return jnp.concatenate([
                dot_hi(lat_ref[0:64, :], W[0:16, :]),
                dot_hi(lon_ref[0:64, :], W[17:33, :]),
                dot_hi(date_ref[0:64, :], W[34:50, :]),
                dot_hi(sst_ref[0:64, :], W[50:66, :]),
                W[16:17, :], W[33:34, :], W[66:67, :], W[67:68, :],
            ], axis=0) * k

        T1_s[...] = tmat(W1_ref).astype(_BF)
        T3_s[...] = tmat(W3_ref).astype(_BF)
        W2_s[...] = (W2_ref[:] * k).astype(_BF)
        W3b_s[...] = (W3_ref[68:196, :] * k).astype(_BF)
        W4_s[...] = W4_ref[:].astype(_BF)
        b4_s[0, :] = b4_ref[:]
        c1_s[0, :] = (rb1_ref[:] - b1_ref[:]) * k
        c2_s[0, :] = (rb2_ref[:] - b2_ref[:]) * k
        c3_s[0, :] = (rb3_ref[:] - b3_ref[:]) * k

    xb = x_ref[:]
    Bb = xb.shape[0]
    iot = jax.lax.broadcasted_iota(jnp.int32, (Bb, 128), 1).astype(_BF)

    def idx_frac(col):
        v = xb[:, col:col + 1]
        fl = jnp.floor(v)
        return fl.astype(_BF), (v - fl).astype(_BF)

    i0, f0 = idx_frac(0)
    i1, f1 = idx_frac(1)
    i2, _ = idx_frac(2)
    i3, f3 = idx_frac(3)

    one = jnp.ones((), _BF)
    zero = jnp.zeros((), _BF)
    oh01 = jnp.where((iot == i0) | (iot == i1 + 64), one, zero)
    oh23 = jnp.where((iot == i2) | (iot == i3 + 64), one, zero)
    fr = jnp.concatenate([f0, f1, f3, xb[:, 4:5].astype(_BF)], axis=1)
    s = jnp.concatenate([oh01, oh23, fr], axis=1)

    def rbf(a, c_s):
        d = a - c_s[...]
        return jnp.exp2(-(d * d)).astype(_BF)

    h = rbf(dot(s, T1_s[...]), c1_s)
    h = rbf(dot(h, W2_s[...]), c2_s)
    g = rbf(dot(s, T3_s[...]) + dot(h, W3b_s[...]), c3_s)
    out_ref[:] = dot(g, W4_s[...]) + b4_s[...]


def kernel(x, emb_lat, emb_lon, emb_sst, emb_date,
           W1, b1, rb1, W2, b2, rb2, W3, b3, rb3, W4, b4):
    B = x.shape[0]
    Bb = 4096
    f32 = jnp.float32

    full = lambda a: pl.BlockSpec(a.shape, lambda i: (0,) * a.ndim)
    operands = [x, emb_lat, emb_lon, emb_sst, emb_date,
                W1, b1, rb1, W2, b2, rb2, W3, b3, rb3, W4, b4]
    in_specs = [pl.BlockSpec((Bb, 5), lambda i: (i, 0))]
    in_specs += [full(a) for a in operands[1:]]

    return pl.pallas_call(
        _mlpe_block,
        grid=(B // Bb,),
        in_specs=in_specs,
        out_specs=pl.BlockSpec((Bb, 300), lambda i: (i, 0)),
        out_shape=jax.ShapeDtypeStruct((B, 300), f32),
        scratch_shapes=[
            pltpu.VMEM((260, 64), _BF), pltpu.VMEM((260, 64), _BF),
            pltpu.VMEM((64, 128), _BF), pltpu.VMEM((128, 64), _BF),
            pltpu.VMEM((64, 300), _BF), pltpu.VMEM((1, 64), f32),
            pltpu.VMEM((1, 128), f32), pltpu.VMEM((1, 64), f32),
            pltpu.VMEM((1, 300), f32),
        ],
    )(*operands)


# speedup vs baseline: 1.0343x; 1.0343x over previous
import functools

import jax
import jax.numpy as jnp
import numpy as np
from jax.experimental import pallas as pl
from jax.experimental.pallas import tpu as pltpu

_PI = float(np.pi)
_BF = jnp.bfloat16


def _mlpe_block(x_ref, lat_ref, lon_ref, sst_ref, date_ref,
                W1_ref, b1_ref, rb1_ref, W2_ref, b2_ref, rb2_ref,
                W3_ref, b3_ref, rb3_ref, W4_ref, b4_ref, out_ref,
                T13_s, W2_s, W3b_s, W4_s, c1_s, c2_s, c3_s, b4_s):
    f32 = jnp.float32
    dot = functools.partial(jnp.dot, preferred_element_type=f32)
    dot_hi = functools.partial(jnp.dot, precision=jax.lax.Precision.HIGHEST,
                               preferred_element_type=f32)

    k = float(np.sqrt(_PI / np.log(2.0)))

    @pl.when(pl.program_id(0) == 0)
    def _fold():
        def tmat(W_ref):
            W = W_ref[0:68, :]
            return jnp.concatenate([
                dot_hi(lat_ref[0:64, :], W[0:16, :]),
                dot_hi(lon_ref[0:64, :], W[17:33, :]),
                dot_hi(date_ref[0:64, :], W[34:50, :]),
                dot_hi(sst_ref[0:64, :], W[50:66, :]),
                W[16:17, :], W[33:34, :], W[66:67, :], W[67:68, :],
            ], axis=0) * k

        T13_s[...] = jnp.concatenate(
            [tmat(W1_ref), tmat(W3_ref)], axis=1).astype(_BF)
        W2_s[...] = (W2_ref[:] * k).astype(_BF)
        W3b_s[...] = (W3_ref[68:196, :] * k).astype(_BF)
        W4_s[...] = W4_ref[:].astype(_BF)
        b4_s[0, :] = b4_ref[:]
        c1_s[0, :] = (rb1_ref[:] - b1_ref[:]) * k
        c2_s[0, :] = (rb2_ref[:] - b2_ref[:]) * k
        c3_s[0, :] = (rb3_ref[:] - b3_ref[:]) * k

    xb = x_ref[:]
    Bb = xb.shape[0]
    iot = jax.lax.broadcasted_iota(jnp.int32, (Bb, 128), 1).astype(_BF)

    def idx_frac(col):
        v = xb[:, col:col + 1]
        fl = jnp.floor(v)
        return fl.astype(_BF), (v - fl).astype(_BF)

    i0, f0 = idx_frac(0)
    i1, f1 = idx_frac(1)
    i2, _ = idx_frac(2)
    i3, f3 = idx_frac(3)

    one = jnp.ones((), _BF)
    zero = jnp.zeros((), _BF)
    oh01 = jnp.where((iot == i0) | (iot == i1 + 64), one, zero)
    oh23 = jnp.where((iot == i2) | (iot == i3 + 64), one, zero)
    fr = jnp.concatenate([f0, f1, f3, xb[:, 4:5].astype(_BF)], axis=1)
    s = jnp.concatenate([oh01, oh23, fr], axis=1)

    def rbf(a, c_s):
        d = a - c_s[...]
        return jnp.exp2(-(d * d)).astype(_BF)

    a13 = dot(s, T13_s[...])
    h = rbf(a13[:, 0:64], c1_s)
    h = rbf(dot(h, W2_s[...]), c2_s)
    g = rbf(a13[:, 64:128] + dot(h, W3b_s[...]), c3_s)
    out_ref[:] = dot(g, W4_s[...]) + b4_s[...]


def kernel(x, emb_lat, emb_lon, emb_sst, emb_date,
           W1, b1, rb1, W2, b2, rb2, W3, b3, rb3, W4, b4):
    B = x.shape[0]
    Bb = 4096
    f32 = jnp.float32

    full = lambda a: pl.BlockSpec(a.shape, lambda i: (0,) * a.ndim)
    operands = [x, emb_lat, emb_lon, emb_sst, emb_date,
                W1, b1, rb1, W2, b2, rb2, W3, b3, rb3, W4, b4]
    in_specs = [pl.BlockSpec((Bb, 5), lambda i: (i, 0))]
    in_specs += [full(a) for a in operands[1:]]

    return pl.pallas_call(
        _mlpe_block,
        grid=(B // Bb,),
        in_specs=in_specs,
        out_specs=pl.BlockSpec((Bb, 300), lambda i: (i, 0)),
        out_shape=jax.ShapeDtypeStruct((B, 300), f32),
        scratch_shapes=[
            pltpu.VMEM((260, 128), _BF),
            pltpu.VMEM((64, 128), _BF), pltpu.VMEM((128, 64), _BF),
            pltpu.VMEM((64, 300), _BF), pltpu.VMEM((1, 64), f32),
            pltpu.VMEM((1, 128), f32), pltpu.VMEM((1, 64), f32),
            pltpu.VMEM((1, 300), f32),
        ],
    )(*operands)
